# Initial kernel scaffold; baseline (speedup 1.0000x reference)
import jax
import jax.numpy as jnp
from jax.experimental import pallas as pl


def kernel(features, edge_index, W0, b0, W1, b1, W2, b2, norm):
    raise NotImplementedError("write your pallas kernel here")


# speedup vs baseline: 2.7081x; 2.7081x over previous
import functools

import jax
import jax.numpy as jnp
from jax import lax
from jax.experimental import pallas as pl
from jax.experimental.pallas import tpu as pltpu
from jax.experimental.pallas import tpu_sc as plsc

N = 10000
E = 160000
IN_FEATS = 512
HID = 512
NCL = 128
K = 10
ALPHA = 0.1

NPAD = 10240
EPAD = 163840
CH = 128
NCHUNK = EPAD // 32 // CH

_info = plsc.get_sparse_core_info()
NC = _info.num_cores
NS = _info.num_subcores
NW = NC * NS
RPT = NPAD // NS

RB = 1024
RB2 = 1280


def _mlp_body(x_ref, w0_ref, b0_ref, w1_ref, b1_ref, w2_ref, b2_ref, n_ref,
              u0_ref, c1_ref, c2_ref):
    h = jnp.dot(x_ref[...], w0_ref[...], preferred_element_type=jnp.float32)
    h = jnp.maximum(h + b0_ref[...], 0.0)
    h = jnp.dot(h, w1_ref[...], preferred_element_type=jnp.float32)
    h = jnp.maximum(h + b1_ref[...], 0.0)
    h = jnp.dot(h, w2_ref[...], preferred_element_type=jnp.float32)
    h = h + b2_ref[...]
    nn = n_ref[...]
    u0_ref[...] = nn * h
    c1_ref[...] = (ALPHA * nn) * h
    c2_ref[...] = ALPHA * h


_mlp = pl.pallas_call(
    _mlp_body,
    grid=(NPAD // RB,),
    in_specs=[
        pl.BlockSpec((RB, IN_FEATS), lambda i: (i, 0)),
        pl.BlockSpec((IN_FEATS, HID), lambda i: (0, 0)),
        pl.BlockSpec((1, HID), lambda i: (0, 0)),
        pl.BlockSpec((HID, HID), lambda i: (0, 0)),
        pl.BlockSpec((1, HID), lambda i: (0, 0)),
        pl.BlockSpec((HID, NCL), lambda i: (0, 0)),
        pl.BlockSpec((1, NCL), lambda i: (0, 0)),
        pl.BlockSpec((RB, 1), lambda i: (i, 0)),
    ],
    out_specs=[pl.BlockSpec((RB, NCL), lambda i: (i, 0))] * 3,
    out_shape=[jax.ShapeDtypeStruct((NPAD, NCL), jnp.float32)] * 3,
)


def _comb_body(s_ref, a_ref, c_ref, o_ref):
    o_ref[...] = a_ref[...] * (s_ref[0] + s_ref[1]) + c_ref[...]


_combine = pl.pallas_call(
    _comb_body,
    grid=(NPAD // RB2,),
    in_specs=[
        pl.BlockSpec((2, RB2, NCL), lambda i: (0, i, 0)),
        pl.BlockSpec((RB2, 1), lambda i: (i, 0)),
        pl.BlockSpec((RB2, NCL), lambda i: (i, 0)),
    ],
    out_specs=pl.BlockSpec((RB2, NCL), lambda i: (i, 0)),
    out_shape=jax.ShapeDtypeStruct((NPAD, NCL), jnp.float32),
)


@functools.partial(
    pl.kernel,
    out_type=jax.ShapeDtypeStruct((NC, NPAD, NCL), jnp.float32),
    mesh=plsc.VectorSubcoreMesh(core_axis_name="c", subcore_axis_name="s"),
    scratch_types=[
        pltpu.VMEM((NCHUNK, CH), jnp.int32),
        pltpu.VMEM((NCHUNK, CH), jnp.int32),
        pltpu.VMEM((CH, NCL), jnp.float32),
        pltpu.VMEM((CH, NCL), jnp.float32),
        pltpu.VMEM_SHARED((NPAD, NCL), jnp.float32),
        pltpu.SemaphoreType.DMA,
    ],
)
def _sc_scatter(u_hbm, src_hbm, dst_hbm, zeros_hbm, out_hbm,
                src_v, dst_v, gbuf, zbuf, acc, sem):
    c = lax.axis_index("c")
    s = lax.axis_index("s")
    wid = s * NC + c

    pltpu.sync_copy(zeros_hbm, zbuf)
    pltpu.sync_copy(src_hbm.at[wid], src_v)
    pltpu.sync_copy(dst_hbm.at[wid], dst_v)

    for z in range(RPT // CH):
        pltpu.sync_copy(zbuf, acc.at[pl.ds(s * RPT + z * CH, CH)])
    plsc.subcore_barrier()

    def body(j, carry):
        pltpu.async_copy(u_hbm.at[src_v.at[j]], gbuf, sem).wait()
        pltpu.sync_copy(gbuf, acc.at[dst_v.at[j]], add=True)
        return carry

    lax.fori_loop(0, NCHUNK, body, 0)
    plsc.subcore_barrier()

    pltpu.sync_copy(acc.at[pl.ds(s * RPT, RPT)],
                    out_hbm.at[c, pl.ds(s * RPT, RPT)])


def kernel(features, edge_index, W0, b0, W1, b1, W2, b2, norm):
    feats_p = jnp.pad(features, ((0, NPAD - N), (0, 0)))
    norm_p = jnp.pad(norm, ((0, NPAD - N), (0, 0)))
    ei_p = jnp.pad(edge_index, ((0, 0), (0, EPAD - E)), constant_values=N)
    src_pk = ei_p[0].reshape(NW, NCHUNK, CH)
    dst_pk = ei_p[1].reshape(NW, NCHUNK, CH)
    zeros = jnp.zeros((CH, NCL), jnp.float32)
    a1 = (1.0 - ALPHA) * norm_p * norm_p
    a2 = (1.0 - ALPHA) * norm_p

    u, c1, c2 = _mlp(feats_p, W0, b0.reshape(1, HID), W1, b1.reshape(1, HID),
                     W2, b2.reshape(1, NCL), norm_p)
    for _ in range(K - 1):
        s_part = _sc_scatter(u, src_pk, dst_pk, zeros)
        u = _combine(s_part, a1, c1)
    s_part = _sc_scatter(u, src_pk, dst_pk, zeros)
    h = _combine(s_part, a2, c2)
    return h[:N]


# speedup vs baseline: 2.9062x; 1.0731x over previous
import functools

import jax
import jax.numpy as jnp
from jax import lax
from jax.experimental import pallas as pl
from jax.experimental.pallas import tpu as pltpu
from jax.experimental.pallas import tpu_sc as plsc

N = 10000
E = 160000
IN_FEATS = 512
HID = 512
NCL = 128
K = 10
ALPHA = 0.1

NPAD = 10240
EPAD = 163840
CH = 128
NCHUNK = EPAD // 32 // CH

_info = plsc.get_sparse_core_info()
NC = _info.num_cores
NS = _info.num_subcores
NW = NC * NS
RPT = NPAD // NS

RB = 1024
RB2 = 1280


def _mlp_body(x_ref, w0_ref, b0_ref, w1_ref, b1_ref, w2_ref, b2_ref, n_ref,
              u0_ref, c1_ref, c2_ref):
    h = jnp.dot(x_ref[...], w0_ref[...], preferred_element_type=jnp.float32)
    h = jnp.maximum(h + b0_ref[...], 0.0)
    h = jnp.dot(h, w1_ref[...], preferred_element_type=jnp.float32)
    h = jnp.maximum(h + b1_ref[...], 0.0)
    h = jnp.dot(h, w2_ref[...], preferred_element_type=jnp.float32)
    h = h + b2_ref[...]
    nn = n_ref[...]
    u0_ref[...] = nn * h
    c1_ref[...] = (ALPHA * nn) * h
    c2_ref[...] = ALPHA * h


_mlp = pl.pallas_call(
    _mlp_body,
    grid=(NPAD // RB,),
    in_specs=[
        pl.BlockSpec((RB, IN_FEATS), lambda i: (i, 0)),
        pl.BlockSpec((IN_FEATS, HID), lambda i: (0, 0)),
        pl.BlockSpec((1, HID), lambda i: (0, 0)),
        pl.BlockSpec((HID, HID), lambda i: (0, 0)),
        pl.BlockSpec((1, HID), lambda i: (0, 0)),
        pl.BlockSpec((HID, NCL), lambda i: (0, 0)),
        pl.BlockSpec((1, NCL), lambda i: (0, 0)),
        pl.BlockSpec((RB, 1), lambda i: (i, 0)),
    ],
    out_specs=[pl.BlockSpec((RB, NCL), lambda i: (i, 0))] * 3,
    out_shape=[jax.ShapeDtypeStruct((NPAD, NCL), jnp.float32)] * 3,
)


def _comb_body(s_ref, a_ref, c_ref, o_ref):
    o_ref[...] = a_ref[...] * (s_ref[0] + s_ref[1]) + c_ref[...]


_combine = pl.pallas_call(
    _comb_body,
    grid=(NPAD // RB2,),
    in_specs=[
        pl.BlockSpec((2, RB2, NCL), lambda i: (0, i, 0)),
        pl.BlockSpec((RB2, 1), lambda i: (i, 0)),
        pl.BlockSpec((RB2, NCL), lambda i: (i, 0)),
    ],
    out_specs=pl.BlockSpec((RB2, NCL), lambda i: (i, 0)),
    out_shape=jax.ShapeDtypeStruct((NPAD, NCL), jnp.float32),
)


@functools.partial(
    pl.kernel,
    out_type=jax.ShapeDtypeStruct((NC, NPAD, NCL), jnp.float32),
    mesh=plsc.VectorSubcoreMesh(core_axis_name="c", subcore_axis_name="s"),
    scratch_types=[
        pltpu.VMEM((NCHUNK, CH), jnp.int32),
        pltpu.VMEM((NCHUNK, CH), jnp.int32),
        pltpu.VMEM((CH, NCL), jnp.float32),
        pltpu.VMEM((CH, NCL), jnp.float32),
        pltpu.VMEM_SHARED((NPAD, NCL), jnp.float32),
        pltpu.SemaphoreType.DMA,
        pltpu.SemaphoreType.DMA,
    ],
)
def _sc_scatter(u_hbm, src_hbm, dst_hbm, zeros_hbm, out_hbm,
                src_v, dst_v, gbuf_a, gbuf_b, acc, sem_a, sem_b):
    c = lax.axis_index("c")
    s = lax.axis_index("s")
    wid = s * NC + c

    pltpu.sync_copy(zeros_hbm, gbuf_a)
    pltpu.sync_copy(src_hbm.at[wid], src_v)
    pltpu.sync_copy(dst_hbm.at[wid], dst_v)

    for z in range(RPT // CH):
        pltpu.sync_copy(gbuf_a, acc.at[pl.ds(s * RPT + z * CH, CH)])
    plsc.subcore_barrier()

    bufs = (gbuf_a, gbuf_b)
    sems = (sem_a, sem_b)
    pltpu.async_copy(u_hbm.at[src_v.at[0]], gbuf_a, sem_a)

    def body(i, carry):
        j = i * 2
        for k in range(2):
            buf, sem = bufs[k], sems[k]
            nbuf, nsem = bufs[1 - k], sems[1 - k]
            nxt = jnp.minimum(j + k + 1, NCHUNK - 1)
            pltpu.async_copy(u_hbm.at[src_v.at[nxt]], nbuf, nsem)
            pltpu.make_async_copy(u_hbm.at[src_v.at[0]], buf, sem).wait()
            pltpu.sync_copy(buf, acc.at[dst_v.at[j + k]], add=True)
        return carry

    lax.fori_loop(0, NCHUNK // 2, body, 0)
    pltpu.make_async_copy(u_hbm.at[src_v.at[0]], bufs[0], sems[0]).wait()
    plsc.subcore_barrier()

    pltpu.sync_copy(acc.at[pl.ds(s * RPT, RPT)],
                    out_hbm.at[c, pl.ds(s * RPT, RPT)])


def kernel(features, edge_index, W0, b0, W1, b1, W2, b2, norm):
    feats_p = jnp.pad(features, ((0, NPAD - N), (0, 0)))
    norm_p = jnp.pad(norm, ((0, NPAD - N), (0, 0)))
    ei_p = jnp.pad(edge_index, ((0, 0), (0, EPAD - E)), constant_values=N)
    src_pk = ei_p[0].reshape(NW, NCHUNK, CH)
    dst_pk = ei_p[1].reshape(NW, NCHUNK, CH)
    zeros = jnp.zeros((CH, NCL), jnp.float32)
    a1 = (1.0 - ALPHA) * norm_p * norm_p
    a2 = (1.0 - ALPHA) * norm_p

    u, c1, c2 = _mlp(feats_p, W0, b0.reshape(1, HID), W1, b1.reshape(1, HID),
                     W2, b2.reshape(1, NCL), norm_p)
    for _ in range(K - 1):
        s_part = _sc_scatter(u, src_pk, dst_pk, zeros)
        u = _combine(s_part, a1, c1)
    s_part = _sc_scatter(u, src_pk, dst_pk, zeros)
    h = _combine(s_part, a2, c2)
    return h[:N]
